# baseline (speedup 1.0000x reference)
import functools

import jax
import jax.numpy as jnp
from jax import lax
from jax.experimental import pallas as pl
from jax.experimental.pallas import tpu as pltpu
from jax.experimental.pallas import tpu_sc as plsc

S0 = 32
S1 = 8
NH = 64


def _gather_rows(emb, x):
    V, D = emb.shape
    B = x.shape[0]
    NC, NS = 2, 16
    NW = NC * NS
    bpw = B // NW

    mesh = plsc.VectorSubcoreMesh(core_axis_name="c", subcore_axis_name="s")

    @functools.partial(
        pl.kernel,
        mesh=mesh,
        out_type=jax.ShapeDtypeStruct((B, D), jnp.float32),
        scratch_types=[
            pltpu.VMEM((bpw,), jnp.int32),
            pltpu.VMEM((bpw, D), jnp.float32),
            pltpu.SemaphoreType.DMA,
        ],
    )
    def k(table_hbm, idx_hbm, out_hbm, idx_v, rows_v, sem):
        wid = lax.axis_index("s") * NC + lax.axis_index("c")
        base = wid * bpw
        pltpu.sync_copy(idx_hbm.at[pl.ds(base, bpw)], idx_v)
        cps = []
        for j in range(bpw // 128):
            cps.append(pltpu.async_copy(
                table_hbm.at[idx_v.at[pl.ds(j * 128, 128)]],
                rows_v.at[pl.ds(j * 128, 128)],
                sem))
        for c in cps:
            c.wait()
        pltpu.sync_copy(rows_v, out_hbm.at[pl.ds(base, bpw)])

    return k(emb, x)


def _project(h, W1, b1, W2, b2, bm):
    n_rows, d_in = h.shape
    d_out = W1.shape[1]

    def body(h_ref, w1_ref, b1_ref, w2_ref, b2_ref, m_ref, n_ref):
        hb = h_ref[...]
        m_ref[...] = jnp.dot(hb, w1_ref[...],
                             preferred_element_type=jnp.float32) + b1_ref[...]
        n_ref[...] = jnp.dot(hb, w2_ref[...],
                             preferred_element_type=jnp.float32) + b2_ref[...]

    return pl.pallas_call(
        body,
        grid=(n_rows // bm,),
        in_specs=[
            pl.BlockSpec((bm, d_in), lambda i: (i, 0)),
            pl.BlockSpec((d_in, d_out), lambda i: (0, 0)),
            pl.BlockSpec((1, d_out), lambda i: (0, 0)),
            pl.BlockSpec((d_in, d_out), lambda i: (0, 0)),
            pl.BlockSpec((1, d_out), lambda i: (0, 0)),
        ],
        out_specs=[
            pl.BlockSpec((bm, d_out), lambda i: (i, 0)),
            pl.BlockSpec((bm, d_out), lambda i: (i, 0)),
        ],
        out_shape=[jax.ShapeDtypeStruct((n_rows, d_out), jnp.float32)] * 2,
    )(h, W1, b1, W2, b2)


def _mp_layer(adj, m, n, attW, attb, attu, bm, nxt=None):
    n_rows = adj.shape[0]
    ns = bm // S0
    full = lambda r, c: pl.BlockSpec((r, c), lambda i: (0, 0))

    def body(*refs):
        if nxt is None:
            (a0, a1, a2, a3, m_ref, n_ref, aw_ref, ab_ref, au_ref,
             pooled_ref) = refs
            adj_refs = (a0, a1, a2, a3)
        else:
            (a0, a1, a2, a3, m_ref, n_ref, aw_ref, ab_ref, au_ref,
             w1_ref, b1_ref, w2_ref, b2_ref,
             pooled_ref, mn_ref, nn_ref) = refs
            adj_refs = (a0, a1, a2, a3)
        hk = n_rows // 4
        acc = sum(
            jnp.dot(adj_refs[j][...], m_ref[pl.ds(j * hk, hk), :],
                    preferred_element_type=jnp.float32)
            for j in range(4))
        h = jnp.maximum(acc + n_ref[...], 0.0)
        t = jnp.tanh(jnp.dot(h, aw_ref[...],
                             preferred_element_type=jnp.float32) + ab_ref[...])
        a = jnp.sum(t * au_ref[...], axis=1, keepdims=True)
        e = jnp.exp(a - jnp.max(a))
        rows = lax.broadcasted_iota(jnp.int32, (ns, bm), 0)
        cols = lax.broadcasted_iota(jnp.int32, (ns, bm), 1)
        seg = jnp.where(cols // S0 == rows, 1.0, 0.0)
        ssum = jnp.dot(seg, e, preferred_element_type=jnp.float32)
        pw = jnp.dot(seg, e * h, preferred_element_type=jnp.float32)
        pooled_ref[...] = pw / ssum
        if nxt is not None:
            mn_ref[...] = jnp.dot(h, w1_ref[...],
                                  preferred_element_type=jnp.float32) + b1_ref[...]
            nn_ref[...] = jnp.dot(h, w2_ref[...],
                                  preferred_element_type=jnp.float32) + b2_ref[...]

    in_specs = [
        pl.BlockSpec((bm, n_rows // 4), lambda i: (i, 0)),
        pl.BlockSpec((bm, n_rows // 4), lambda i: (i, 1)),
        pl.BlockSpec((bm, n_rows // 4), lambda i: (i, 2)),
        pl.BlockSpec((bm, n_rows // 4), lambda i: (i, 3)),
        full(n_rows, NH),
        pl.BlockSpec((bm, NH), lambda i: (i, 0)),
        full(NH, NH),
        full(1, NH),
        full(1, NH),
    ]
    out_specs = [pl.BlockSpec((ns, NH), lambda i: (i, 0))]
    out_shape = [jax.ShapeDtypeStruct((n_rows // S0, NH), jnp.float32)]
    args = [adj, adj, adj, adj, m, n, attW, attb, attu]
    if nxt is not None:
        w1n, b1n, w2n, b2n = nxt
        in_specs += [full(NH, NH), full(1, NH), full(NH, NH), full(1, NH)]
        out_specs += [pl.BlockSpec((bm, NH), lambda i: (i, 0))] * 2
        out_shape += [jax.ShapeDtypeStruct((n_rows, NH), jnp.float32)] * 2
        args += [w1n, b1n, w2n, b2n]

    return pl.pallas_call(
        body,
        grid=(n_rows // bm,),
        in_specs=in_specs,
        out_specs=out_specs,
        out_shape=out_shape,
        compiler_params=pltpu.CompilerParams(
            dimension_semantics=("arbitrary",)),
    )(*args)


def _head(p0, p1, g0, b0, g1, b1, w1a, w1b, fb1, aw, ab, au, w2, fb2, w3, fb3):
    n_sent = p0.shape[0]
    n_doc = n_sent // S1
    nc = w3.shape[1]

    def body(p0_ref, p1_ref, g0_ref, b0_ref, g1_ref, b1_ref, w1a_ref, w1b_ref,
             fb1_ref, aw_ref, ab_ref, au_ref, w2_ref, fb2_ref, w3_ref, fb3_ref,
             out_ref):
        z0 = p0_ref[...]
        z1 = p1_ref[...]
        mu0 = jnp.mean(z0, axis=0, keepdims=True)
        v0 = jnp.mean((z0 - mu0) ** 2, axis=0, keepdims=True)
        z0 = (z0 - mu0) * lax.rsqrt(v0 + 1e-5) * g0_ref[...] + b0_ref[...]
        mu1 = jnp.mean(z1, axis=0, keepdims=True)
        v1 = jnp.mean((z1 - mu1) ** 2, axis=0, keepdims=True)
        z1 = (z1 - mu1) * lax.rsqrt(v1 + 1e-5) * g1_ref[...] + b1_ref[...]
        zf = jnp.maximum(
            jnp.dot(z0, w1a_ref[...], preferred_element_type=jnp.float32)
            + jnp.dot(z1, w1b_ref[...], preferred_element_type=jnp.float32)
            + fb1_ref[...], 0.0)
        t = jnp.tanh(jnp.dot(zf, aw_ref[...],
                             preferred_element_type=jnp.float32) + ab_ref[...])
        a = jnp.sum(t * au_ref[...], axis=1, keepdims=True)
        e = jnp.exp(a - jnp.max(a))
        rows = lax.broadcasted_iota(jnp.int32, (n_doc, n_sent), 0)
        cols = lax.broadcasted_iota(jnp.int32, (n_doc, n_sent), 1)
        seg = jnp.where(cols // S1 == rows, 1.0, 0.0)
        ssum = jnp.dot(seg, e, preferred_element_type=jnp.float32)
        pw = jnp.dot(seg, e * zf, preferred_element_type=jnp.float32)
        zs = pw / ssum
        z2 = jnp.maximum(
            jnp.dot(zs, w2_ref[...], preferred_element_type=jnp.float32)
            + fb2_ref[...], 0.0)
        z3 = jnp.dot(z2, w3_ref[...],
                     preferred_element_type=jnp.float32) + fb3_ref[...]
        mx = jnp.max(z3, axis=1, keepdims=True)
        lse = jnp.log(jnp.sum(jnp.exp(z3 - mx), axis=1, keepdims=True))
        out_ref[...] = z3 - mx - lse

    full = lambda shape: pl.BlockSpec(shape, lambda: (0,) * len(shape))
    args = (p0, p1, g0, b0, g1, b1, w1a, w1b, fb1, aw, ab, au, w2, fb2, w3, fb3)
    return pl.pallas_call(
        body,
        in_specs=[full(a.shape) for a in args],
        out_specs=full((n_doc, nc)),
        out_shape=jax.ShapeDtypeStruct((n_doc, nc), jnp.float32),
    )(*args)


def kernel(x, adj, adj_s, shapes, emb, params):
    h = _gather_rows(emb, x)
    m0, n0 = _project(
        h,
        params['mp0_W1'], params['mp0_b1'].reshape(1, NH),
        params['mp0_W2'], params['mp0_b2'].reshape(1, NH),
        bm=1024)
    p0, m1, n1 = _mp_layer(
        adj, m0, n0,
        params['att0_W'], params['att0_b'].reshape(1, NH),
        params['att0_u'].reshape(1, NH),
        bm=512,
        nxt=(params['mp1_W1'], params['mp1_b1'].reshape(1, NH),
             params['mp1_W2'], params['mp1_b2'].reshape(1, NH)))
    (p1,) = _mp_layer(
        adj, m1, n1,
        params['att1_W'], params['att1_b'].reshape(1, NH),
        params['att1_u'].reshape(1, NH),
        bm=512)
    fc1_W = params['fc1_W']
    return _head(
        p0, p1,
        params['bn_g'][:NH].reshape(1, NH), params['bn_b'][:NH].reshape(1, NH),
        params['bn_g'][NH:].reshape(1, NH), params['bn_b'][NH:].reshape(1, NH),
        fc1_W[:NH], fc1_W[NH:], params['fc1_b'].reshape(1, NH),
        params['attS_W'], params['attS_b'].reshape(1, NH),
        params['attS_u'].reshape(1, NH),
        params['fc2_W'], params['fc2_b'].reshape(1, NH),
        params['fc3_W'], params['fc3_b'].reshape(1, -1))


# speedup vs baseline: 1.0540x; 1.0540x over previous
import functools

import jax
import jax.numpy as jnp
from jax import lax
from jax.experimental import pallas as pl
from jax.experimental.pallas import tpu as pltpu
from jax.experimental.pallas import tpu_sc as plsc

S0 = 32
S1 = 8
NH = 64
NSPLIT = 4


def _gather_rows(emb, x):
    V, D = emb.shape
    B = x.shape[0]
    NC, NS = 2, 16
    NW = NC * NS
    bpw = B // NW

    mesh = plsc.VectorSubcoreMesh(core_axis_name="c", subcore_axis_name="s")

    @functools.partial(
        pl.kernel,
        mesh=mesh,
        out_type=jax.ShapeDtypeStruct((B, D), jnp.float32),
        scratch_types=[
            pltpu.VMEM((bpw,), jnp.int32),
            pltpu.VMEM((bpw, D), jnp.float32),
            pltpu.SemaphoreType.DMA,
        ],
    )
    def k(table_hbm, idx_hbm, out_hbm, idx_v, rows_v, sem):
        wid = lax.axis_index("s") * NC + lax.axis_index("c")
        base = wid * bpw
        pltpu.sync_copy(idx_hbm.at[pl.ds(base, bpw)], idx_v)
        cps = []
        for j in range(bpw // 128):
            cps.append(pltpu.async_copy(
                table_hbm.at[idx_v.at[pl.ds(j * 128, 128)]],
                rows_v.at[pl.ds(j * 128, 128)],
                sem))
        for c in cps:
            c.wait()
        pltpu.sync_copy(rows_v, out_hbm.at[pl.ds(base, bpw)])

    return k(emb, x)


def _project(h, W1, b1, W2, b2, bm):
    n_rows, d_in = h.shape
    d_out = W1.shape[1]

    def body(h_ref, w1_ref, b1_ref, w2_ref, b2_ref, m_ref, n_ref):
        hb = h_ref[...]
        m_ref[...] = jnp.dot(hb, w1_ref[...],
                             preferred_element_type=jnp.float32) + b1_ref[...]
        n_ref[...] = jnp.dot(hb, w2_ref[...],
                             preferred_element_type=jnp.float32) + b2_ref[...]

    return pl.pallas_call(
        body,
        grid=(n_rows // bm,),
        in_specs=[
            pl.BlockSpec((bm, d_in), lambda i: (i, 0)),
            pl.BlockSpec((d_in, d_out), lambda i: (0, 0)),
            pl.BlockSpec((1, d_out), lambda i: (0, 0)),
            pl.BlockSpec((d_in, d_out), lambda i: (0, 0)),
            pl.BlockSpec((1, d_out), lambda i: (0, 0)),
        ],
        out_specs=[
            pl.BlockSpec((bm, d_out), lambda i: (i, 0)),
            pl.BlockSpec((bm, d_out), lambda i: (i, 0)),
        ],
        out_shape=[jax.ShapeDtypeStruct((n_rows, d_out), jnp.float32)] * 2,
    )(h, W1, b1, W2, b2)


def _seg_softmax_pool(h, aw, ab, au, seg_len):
    bm = h.shape[0]
    ns = bm // seg_len
    t = jnp.tanh(jnp.dot(h, aw, preferred_element_type=jnp.float32) + ab)
    a = jnp.sum(t * au, axis=1, keepdims=True)
    e = jnp.exp(a - jnp.max(a))
    rows = lax.broadcasted_iota(jnp.int32, (ns, bm), 0)
    cols = lax.broadcasted_iota(jnp.int32, (ns, bm), 1)
    seg = jnp.where(cols // seg_len == rows, 1.0, 0.0)
    ssum = jnp.dot(seg, e, preferred_element_type=jnp.float32)
    pw = jnp.dot(seg, e * h, preferred_element_type=jnp.float32)
    return pw / ssum


def _fused_mp(adj, m0, n0, attWs, attbs, attus, w1n, b1n, w2n, b2n,
              bn_g, bn_b, fc1_W, fc1_b, attS_W, attS_b, attS_u,
              fc2_W, fc2_b, fc3_W, fc3_b, bm):
    n_rows = adj.shape[0]
    NI = n_rows // bm
    ns = bm // S0
    n_sent = n_rows // S0
    n_doc = n_sent // S1
    nc = fc3_W.shape[1]
    pk = n_rows // NSPLIT
    full = lambda *shape: pl.BlockSpec(shape, lambda l, i: (0,) * len(shape))

    def body(a0, a1, a2, a3, m_ref, n_ref, aw_ref, ab_ref, au_ref,
             w1_ref, b1_ref, w2_ref, b2_ref,
             g_ref, bb_ref, w1h_ref, b1h_ref, asw_ref, asb_ref, asu_ref,
             w2h_ref, b2h_ref, w3h_ref, b3h_ref,
             out_ref, m1_s, n1_s, pool_s):
        l, i = pl.program_id(0), pl.program_id(1)
        adj_refs = (a0, a1, a2, a3)

        @pl.when(l == 0)
        def _():
            acc = sum(
                jnp.dot(adj_refs[j][...], m_ref[pl.ds(j * pk, pk), :],
                        preferred_element_type=jnp.float32)
                for j in range(NSPLIT))
            h = jnp.maximum(acc + n_ref[...], 0.0)
            pool_s[pl.ds(i * ns, ns), :] = _seg_softmax_pool(
                h, aw_ref[0], ab_ref[0], au_ref[0], S0)
            m1_s[pl.ds(i * bm, bm), :] = jnp.dot(
                h, w1_ref[...], preferred_element_type=jnp.float32) + b1_ref[...]
            n1_s[pl.ds(i * bm, bm), :] = jnp.dot(
                h, w2_ref[...], preferred_element_type=jnp.float32) + b2_ref[...]

        @pl.when(l == 1)
        def _():
            acc = sum(
                jnp.dot(adj_refs[j][...], m1_s[pl.ds(j * pk, pk), :],
                        preferred_element_type=jnp.float32)
                for j in range(NSPLIT))
            h = jnp.maximum(acc + n1_s[pl.ds(i * bm, bm), :], 0.0)
            pool_s[pl.ds(n_sent + i * ns, ns), :] = _seg_softmax_pool(
                h, aw_ref[0], ab_ref[0], au_ref[0], S0)

        @pl.when((l == 1) & (i == NI - 1))
        def _():
            z0 = pool_s[pl.ds(0, n_sent), :]
            z1 = pool_s[pl.ds(n_sent, n_sent), :]
            mu0 = jnp.mean(z0, axis=0, keepdims=True)
            v0 = jnp.mean((z0 - mu0) ** 2, axis=0, keepdims=True)
            z0 = ((z0 - mu0) * lax.rsqrt(v0 + 1e-5) * g_ref[:, pl.ds(0, NH)]
                  + bb_ref[:, pl.ds(0, NH)])
            mu1 = jnp.mean(z1, axis=0, keepdims=True)
            v1 = jnp.mean((z1 - mu1) ** 2, axis=0, keepdims=True)
            z1 = ((z1 - mu1) * lax.rsqrt(v1 + 1e-5) * g_ref[:, pl.ds(NH, NH)]
                  + bb_ref[:, pl.ds(NH, NH)])
            zf = jnp.maximum(
                jnp.dot(z0, w1h_ref[pl.ds(0, NH), :],
                        preferred_element_type=jnp.float32)
                + jnp.dot(z1, w1h_ref[pl.ds(NH, NH), :],
                          preferred_element_type=jnp.float32)
                + b1h_ref[...], 0.0)
            zs = _seg_softmax_pool(zf, asw_ref[...], asb_ref[...],
                                   asu_ref[...], S1)
            z2 = jnp.maximum(
                jnp.dot(zs, w2h_ref[...], preferred_element_type=jnp.float32)
                + b2h_ref[...], 0.0)
            z3 = jnp.dot(z2, w3h_ref[...],
                         preferred_element_type=jnp.float32) + b3h_ref[...]
            mx = jnp.max(z3, axis=1, keepdims=True)
            lse = jnp.log(jnp.sum(jnp.exp(z3 - mx), axis=1, keepdims=True))
            out_ref[...] = z3 - mx - lse

    in_specs = (
        [pl.BlockSpec((bm, pk), lambda l, i, j=j: (i, j))
         for j in range(NSPLIT)]
        + [pl.BlockSpec((n_rows, NH), lambda l, i: (0, 0)),
           pl.BlockSpec((bm, NH), lambda l, i: ((1 - l) * i, 0)),
           pl.BlockSpec((1, NH, NH), lambda l, i: (l, 0, 0)),
           pl.BlockSpec((1, 1, NH), lambda l, i: (l, 0, 0)),
           pl.BlockSpec((1, 1, NH), lambda l, i: (l, 0, 0)),
           full(NH, NH), full(1, NH), full(NH, NH), full(1, NH),
           full(1, 2 * NH), full(1, 2 * NH),
           full(2 * NH, NH), full(1, NH),
           full(NH, NH), full(1, NH), full(1, NH),
           full(NH, NH), full(1, NH), full(NH, nc), full(1, nc)])

    return pl.pallas_call(
        body,
        grid=(2, NI),
        in_specs=in_specs,
        out_specs=full(n_doc, nc),
        out_shape=jax.ShapeDtypeStruct((n_doc, nc), jnp.float32),
        scratch_shapes=[
            pltpu.VMEM((n_rows, NH), jnp.float32),
            pltpu.VMEM((n_rows, NH), jnp.float32),
            pltpu.VMEM((2 * n_sent, NH), jnp.float32),
        ],
        compiler_params=pltpu.CompilerParams(
            dimension_semantics=("arbitrary", "arbitrary")),
    )(adj, adj, adj, adj, m0, n0, attWs, attbs, attus,
      w1n, b1n, w2n, b2n, bn_g, bn_b, fc1_W, fc1_b,
      attS_W, attS_b, attS_u, fc2_W, fc2_b, fc3_W, fc3_b)


def kernel(x, adj, adj_s, shapes, emb, params):
    h = _gather_rows(emb, x)
    m0, n0 = _project(
        h,
        params['mp0_W1'], params['mp0_b1'].reshape(1, NH),
        params['mp0_W2'], params['mp0_b2'].reshape(1, NH),
        bm=1024)
    attWs = jnp.stack([params['att0_W'], params['att1_W']])
    attbs = jnp.stack([params['att0_b'].reshape(1, NH),
                       params['att1_b'].reshape(1, NH)])
    attus = jnp.stack([params['att0_u'].reshape(1, NH),
                       params['att1_u'].reshape(1, NH)])
    return _fused_mp(
        adj, m0, n0, attWs, attbs, attus,
        params['mp1_W1'], params['mp1_b1'].reshape(1, NH),
        params['mp1_W2'], params['mp1_b2'].reshape(1, NH),
        params['bn_g'].reshape(1, 2 * NH), params['bn_b'].reshape(1, 2 * NH),
        params['fc1_W'], params['fc1_b'].reshape(1, NH),
        params['attS_W'], params['attS_b'].reshape(1, NH),
        params['attS_u'].reshape(1, NH),
        params['fc2_W'], params['fc2_b'].reshape(1, NH),
        params['fc3_W'], params['fc3_b'].reshape(1, -1),
        bm=512)
